# Initial kernel scaffold; baseline (speedup 1.0000x reference)
import jax
import jax.numpy as jnp
from jax.experimental import pallas as pl


def kernel(in_feats, module_ids, W, b):
    raise NotImplementedError("write your pallas kernel here")


# speedup vs baseline: 1.5105x; 1.5105x over previous
import functools

import jax
import jax.numpy as jnp
from jax.experimental import pallas as pl
from jax.experimental.pallas import tpu as pltpu

NUM_MODULES = 8
IN_SIZE = 2048
OUT_SIZE = 2048
NUM_FEATS = 8192

TM = 256
NUM_TILES = NUM_FEATS // TM + NUM_MODULES
PAD_ROWS = NUM_TILES * TM


def _mm_body(tile_expert_ref, num_tiles_ref, x_ref, w_ref, b_ref, o_ref):
    t = pl.program_id(0)

    @pl.when(t < num_tiles_ref[0])
    def _():
        acc = jnp.dot(x_ref[...], w_ref[0],
                      preferred_element_type=jnp.float32)
        o_ref[...] = acc + b_ref[0]


def _grouped_matmul(x_sorted, W, b, tile_expert, num_tiles):
    grid_spec = pltpu.PrefetchScalarGridSpec(
        num_scalar_prefetch=2,
        grid=(NUM_TILES,),
        in_specs=[
            pl.BlockSpec((TM, IN_SIZE), lambda t, te, nt: (t, 0)),
            pl.BlockSpec((1, IN_SIZE, OUT_SIZE), lambda t, te, nt: (te[t], 0, 0)),
            pl.BlockSpec((1, 1, OUT_SIZE), lambda t, te, nt: (te[t], 0, 0)),
        ],
        out_specs=pl.BlockSpec((TM, OUT_SIZE), lambda t, te, nt: (t, 0)),
    )
    return pl.pallas_call(
        _mm_body,
        grid_spec=grid_spec,
        out_shape=jax.ShapeDtypeStruct((PAD_ROWS, OUT_SIZE), jnp.float32),
    )(tile_expert, num_tiles, x_sorted, W, b.reshape(NUM_MODULES, 1, OUT_SIZE))


def kernel(in_feats, module_ids, W, b):
    ids = module_ids.astype(jnp.int32)

    counts = jnp.bincount(ids, length=NUM_MODULES)
    tiles_per_e = (counts + TM - 1) // TM
    start_tile = jnp.concatenate([jnp.zeros((1,), jnp.int32),
                                  jnp.cumsum(tiles_per_e)[:-1].astype(jnp.int32)])
    padded_start = start_tile * TM
    num_tiles = jnp.sum(tiles_per_e).astype(jnp.int32).reshape(1)

    order = jnp.argsort(ids)
    ids_sorted = ids[order]
    group_start = jnp.concatenate([jnp.zeros((1,), jnp.int32),
                                   jnp.cumsum(counts)[:-1].astype(jnp.int32)])
    slot = padded_start[ids_sorted] + (jnp.arange(NUM_FEATS, dtype=jnp.int32)
                                       - group_start[ids_sorted])
    dest = jnp.zeros((NUM_FEATS,), jnp.int32).at[order].set(slot)

    tvec = jnp.arange(NUM_TILES, dtype=jnp.int32)
    tile_expert = (jnp.sum(tvec[:, None] >= start_tile[None, :], axis=1) - 1
                   ).astype(jnp.int32)

    x_bf = in_feats.astype(jnp.bfloat16)
    x_sorted = jnp.zeros((PAD_ROWS, IN_SIZE), jnp.bfloat16).at[dest].set(x_bf)

    out_sorted = _grouped_matmul(x_sorted, W.astype(jnp.bfloat16), b,
                                 tile_expert, num_tiles)

    return out_sorted[dest]


# speedup vs baseline: 1.6520x; 1.0937x over previous
import functools

import jax
import jax.numpy as jnp
from jax.experimental import pallas as pl
from jax.experimental.pallas import tpu as pltpu

NUM_MODULES = 8
IN_SIZE = 2048
OUT_SIZE = 2048
NUM_FEATS = 8192

TM = 256
NUM_TILES = NUM_FEATS // TM + NUM_MODULES
PAD_ROWS = NUM_TILES * TM


def _mm_body(tile_expert_ref, num_tiles_ref, x_ref, w_ref, b_ref, o_ref):
    t = pl.program_id(0)

    @pl.when(t < num_tiles_ref[0])
    def _():
        acc = jnp.dot(x_ref[...], w_ref[0],
                      preferred_element_type=jnp.float32)
        o_ref[...] = acc + b_ref[0]


def _grouped_matmul(x_sorted, W, b, tile_expert, num_tiles):
    grid_spec = pltpu.PrefetchScalarGridSpec(
        num_scalar_prefetch=2,
        grid=(NUM_TILES,),
        in_specs=[
            pl.BlockSpec((TM, IN_SIZE), lambda t, te, nt: (t, 0)),
            pl.BlockSpec((1, IN_SIZE, OUT_SIZE), lambda t, te, nt: (te[t], 0, 0)),
            pl.BlockSpec((1, 1, OUT_SIZE), lambda t, te, nt: (te[t], 0, 0)),
        ],
        out_specs=pl.BlockSpec((TM, OUT_SIZE), lambda t, te, nt: (t, 0)),
    )
    return pl.pallas_call(
        _mm_body,
        grid_spec=grid_spec,
        out_shape=jax.ShapeDtypeStruct((PAD_ROWS, OUT_SIZE), jnp.float32),
    )(tile_expert, num_tiles, x_sorted, W, b.reshape(NUM_MODULES, 1, OUT_SIZE))


def kernel(in_feats, module_ids, W, b):
    ids = module_ids.astype(jnp.int32)

    oh = (ids[:, None] == jnp.arange(NUM_MODULES, dtype=jnp.int32)[None, :]
          ).astype(jnp.int32)
    counts = oh.sum(axis=0)
    tiles_per_e = (counts + TM - 1) // TM
    start_tile = jnp.concatenate([jnp.zeros((1,), jnp.int32),
                                  jnp.cumsum(tiles_per_e)[:-1].astype(jnp.int32)])
    padded_start = start_tile * TM
    num_tiles = jnp.sum(tiles_per_e).astype(jnp.int32).reshape(1)

    rank = jnp.sum((jnp.cumsum(oh, axis=0) - 1) * oh, axis=1)
    dest = padded_start[ids] + rank.astype(jnp.int32)

    tvec = jnp.arange(NUM_TILES, dtype=jnp.int32)
    tile_expert = (jnp.sum(tvec[:, None] >= start_tile[None, :], axis=1) - 1
                   ).astype(jnp.int32)

    x_bf = in_feats.astype(jnp.bfloat16)
    x_sorted = jnp.zeros((PAD_ROWS, IN_SIZE), jnp.bfloat16).at[dest].set(x_bf)

    out_sorted = _grouped_matmul(x_sorted, W.astype(jnp.bfloat16), b,
                                 tile_expert, num_tiles)

    return out_sorted[dest]
